# baseline (speedup 1.0000x reference)
import jax
import jax.numpy as jnp
from jax.experimental import pallas as pl

N = 10000
E = 160000
IN_DIM = 2
UNITS = 64
KDIFF = 2
B = 16
D = IN_DIM + UNITS
M = 2 * KDIFF + 1


def _spmm(rows, cols, vals, X):
    return jax.ops.segment_sum(vals[:, None] * X[cols], rows, num_segments=N)


def _gconv(inputs, state, supports, W, b, out_dim):
    bsz = inputs.shape[0]
    x = jnp.concatenate([inputs.reshape(bsz, N, IN_DIM), state.reshape(bsz, N, UNITS)], axis=2)
    x0 = jnp.transpose(x, (1, 2, 0)).reshape(N, D * bsz)
    xs = [x0]
    for (rows, cols, vals) in supports:
        x1 = _spmm(rows, cols, vals, x0)
        xs.append(x1)
        for k in range(2, KDIFF + 1):
            x2 = 2.0 * _spmm(rows, cols, vals, x1) - x0
            xs.append(x2)
            x1, x0 = x2, x1
    num_matrices = len(xs)
    xcat = jnp.stack(xs, axis=0).reshape(num_matrices, N, D, bsz)
    xcat = jnp.transpose(xcat, (3, 1, 2, 0)).reshape(bsz * N, D * num_matrices)
    out = xcat @ W + b
    return out.reshape(bsz, N * out_dim)


def _gate_kernel(u_ref, hx_ref, c_ref, o_ref):
    u = u_ref[...]
    o_ref[...] = u * hx_ref[...] + (1.0 - u) * jnp.tanh(c_ref[...])


def kernel(inputs, hx, edge_index, edge_weight, W_fn, b_fn, W_gc, b_gc):
    row = edge_index[0]
    col = edge_index[1]
    w = edge_weight
    deg_r = jax.ops.segment_sum(w, row, num_segments=N)
    deg_c = jax.ops.segment_sum(w, col, num_segments=N)
    dinv_r = jnp.where(deg_r > 0, 1.0 / deg_r, 0.0)
    dinv_c = jnp.where(deg_c > 0, 1.0 / deg_c, 0.0)
    supports = [(col, row, w * dinv_r[row]), (row, col, w * dinv_c[col])]

    value = jax.nn.sigmoid(_gconv(inputs, hx, supports, W_fn, b_fn, 2 * UNITS))
    value = value.reshape(-1, N, 2 * UNITS)
    r = value[:, :, :UNITS].reshape(-1, N * UNITS)
    u = value[:, :, UNITS:].reshape(-1, N * UNITS)
    c = _gconv(inputs, r * hx, supports, W_gc, b_gc, UNITS)

    blk = 6400
    grid = (N * UNITS) // blk
    new_state = pl.pallas_call(
        _gate_kernel,
        grid=(1, grid),
        in_specs=[pl.BlockSpec((B, blk), lambda j, i: (j, i))] * 3,
        out_specs=pl.BlockSpec((B, blk), lambda j, i: (j, i)),
        out_shape=jax.ShapeDtypeStruct((B, N * UNITS), jnp.float32),
    )(u, hx, c)
    return new_state


# speedup vs baseline: 1.7963x; 1.7963x over previous
import jax
import jax.numpy as jnp
from jax import lax
from jax.experimental import pallas as pl
from jax.experimental.pallas import tpu as pltpu
from jax.experimental.pallas import tpu_sc as plsc

N = 10000
E = 160000
IN_DIM = 2
UNITS = 64
B = 16
D = IN_DIM + UNITS
M = 5
NSL = 11
SW = (D * B) // NSL
VEC = 16
EPT = E // 16
KC = 80
NCH = EPT // KC
RPT = N // 16
DRC = 125
DEGR = (N + VEC * VEC - 1) // VEC + 1


def _fori(hi, body):
    lax.fori_loop(jnp.int32(0), jnp.int32(hi), lambda i, a: (body(i), a)[1], jnp.int32(0))

def _edge_prep_body(rowh, colh, wh, v1h, v2h, idx_v, w_v, out_v, degr_v, degc_v):
    c = lax.axis_index("c")
    s = lax.axis_index("s")
    wid = s * 2 + c

    def zero_deg(i):
        degr_v[i] = jnp.zeros((VEC,), jnp.float32)
        degc_v[i] = jnp.zeros((VEC,), jnp.float32)

    _fori(640, zero_deg)

    def acc_table(idxh, deg_v):
        def blk(bi):
            pltpu.sync_copy(idxh.at[pl.ds(bi * EPT, EPT)], idx_v)
            pltpu.sync_copy(wh.at[pl.ds(bi * EPT, EPT)], w_v)

            def ed(i):
                o = pl.multiple_of(i * VEC, VEC)
                ids = idx_v[pl.ds(o, VEC)]
                hi = lax.shift_right_logical(ids, jnp.int32(4))
                lo = lax.bitwise_and(ids, jnp.int32(15))
                plsc.addupdate_scatter(deg_v, [hi, lo], w_v[pl.ds(o, VEC)])

            _fori(EPT // VEC, ed)

        _fori(16, blk)

    acc_table(rowh, degr_v)
    acc_table(colh, degc_v)

    def dinv(i):
        dr = degr_v[i]
        degr_v[i] = jnp.where(dr > 0, 1.0 / dr, jnp.zeros((VEC,), jnp.float32))
        dc = degc_v[i]
        degc_v[i] = jnp.where(dc > 0, 1.0 / dc, jnp.zeros((VEC,), jnp.float32))

    _fori(640, dinv)

    def vpass(idxh, deg_v, outh):
        sh = E // 32
        pltpu.sync_copy(idxh.at[pl.ds(wid * sh, sh)], idx_v.at[pl.ds(0, sh)])
        pltpu.sync_copy(wh.at[pl.ds(wid * sh, sh)], w_v.at[pl.ds(0, sh)])

        def ve(i):
            o = pl.multiple_of(i * VEC, VEC)
            ids = idx_v[pl.ds(o, VEC)]
            hi = lax.shift_right_logical(ids, jnp.int32(4))
            lo = lax.bitwise_and(ids, jnp.int32(15))
            dv = plsc.load_gather(deg_v, [hi, lo])
            out_v[pl.ds(o, VEC)] = w_v[pl.ds(o, VEC)] * dv

        _fori((sh + VEC - 1) // VEC, ve)
        pltpu.sync_copy(out_v.at[pl.ds(0, sh)], outh.at[pl.ds(wid * sh, sh)])

    vpass(rowh, degr_v, v1h)
    vpass(colh, degc_v, v2h)


def _build_edge_prep(mesh):
    return pl.kernel(
        _edge_prep_body,
        out_type=(
            jax.ShapeDtypeStruct((E,), jnp.float32),
            jax.ShapeDtypeStruct((E,), jnp.float32),
        ),
        mesh=mesh,
        compiler_params=pltpu.CompilerParams(needs_layout_passes=False, use_tc_tiling_on_sc=False),
        scratch_types=[
            pltpu.VMEM((EPT,), jnp.int32),
            pltpu.VMEM((EPT,), jnp.float32),
            pltpu.VMEM((5008,), jnp.float32),
            pltpu.VMEM((DEGR, VEC), jnp.float32),
            pltpu.VMEM((DEGR, VEC), jnp.float32),
        ],
    )


def _spmm_body(x6, dsth, srch, valsh, y6, cols_v, dst_v, vals_v, rowbuf, big_v, acc, sem):
    c = lax.axis_index("c")
    t = lax.axis_index("s")
    pltpu.sync_copy(srch.at[pl.ds(t * EPT, EPT)], cols_v)
    pltpu.sync_copy(dsth.at[pl.ds(t * NCH, NCH)], dst_v)
    pltpu.sync_copy(valsh.at[pl.ds(t * NCH, NCH)], vals_v)
    for sl in range((NSL + 1) // 2):
        s_idx = c * ((NSL + 1) // 2) + sl
        _spmm_slice(x6, y6, s_idx, t, cols_v, dst_v, vals_v, rowbuf, big_v, acc, sem)


def _spmm_slice(x6, y6, s_idx, t, cols_v, dst_v, vals_v, rowbuf, big_v, acc, sem):
    @pl.when(s_idx < NSL)
    def _():
        def zrow(i):
            for k2 in range(SW // VEC):
                big_v[i, pl.ds(k2 * VEC, VEC)] = jnp.zeros((VEC,), jnp.float32)

        _fori(DRC, zrow)
        for z in range(RPT // DRC):
            pltpu.sync_copy(big_v, acc.at[pl.ds(t * RPT + z * DRC, DRC)])
        plsc.subcore_barrier()

        def chunk(j):
            o = pl.multiple_of(j * KC, KC)
            pltpu.async_copy(
                x6.at[s_idx].at[cols_v.at[pl.ds(o, KC)]], rowbuf, sem
            ).wait()

            def edge(e):
                jv = lax.broadcast(j, (VEC,))
                ev = lax.broadcast(e, (VEC,))
                vs = plsc.load_gather(vals_v, [jv, ev])
                for i2 in range(SW // VEC):
                    rowbuf[e, pl.ds(i2 * VEC, VEC)] = (
                        rowbuf[e, pl.ds(i2 * VEC, VEC)] * vs
                    )

            _fori(KC, edge)
            pltpu.sync_copy(rowbuf, acc.at[dst_v.at[j]], add=True)

        _fori(NCH, chunk)
        plsc.subcore_barrier()
        for z in range(RPT // DRC):
            pltpu.sync_copy(acc.at[pl.ds(t * RPT + z * DRC, DRC)], big_v)
            pltpu.sync_copy(big_v, y6.at[s_idx].at[pl.ds(t * RPT + z * DRC, DRC)])
        plsc.subcore_barrier()


def _build_spmm(mesh):
    return pl.kernel(
        _spmm_body,
        out_type=jax.ShapeDtypeStruct((NSL, N, SW), jnp.float32),
        mesh=mesh,
        compiler_params=pltpu.CompilerParams(needs_layout_passes=False, use_tc_tiling_on_sc=False),
        scratch_types=[
            pltpu.VMEM((EPT,), jnp.int32),
            pltpu.VMEM((NCH, KC), jnp.int32),
            pltpu.VMEM((NCH, KC), jnp.float32),
            pltpu.VMEM((KC, SW), jnp.float32),
            pltpu.VMEM((DRC, SW), jnp.float32),
            pltpu.VMEM_SHARED((N, SW), jnp.float32),
            pltpu.SemaphoreType.DMA,
        ],
    )


_SC_KERNELS = []


def _get_sc_kernels():
    if not _SC_KERNELS:
        mesh = plsc.VectorSubcoreMesh(
            core_axis_name="c", subcore_axis_name="s", num_cores=2, num_subcores=16
        )
        _SC_KERNELS.append((_build_edge_prep(mesh), _build_spmm(mesh)))
    return _SC_KERNELS[0]


def _mix_sig_body(x_ref, w_ref, b_ref, hx_ref, u_ref, rh_ref):
    z = jnp.dot(x_ref[...], w_ref[...], preferred_element_type=jnp.float32)
    v = jax.nn.sigmoid(z + b_ref[0:1, :])
    u_ref[...] = v[:, UNITS:]
    rh_ref[...] = v[:, :UNITS] * hx_ref[...]


def _mix_gate_body(x_ref, w_ref, b_ref, u_ref, hx_ref, o_ref):
    z = jnp.dot(x_ref[...], w_ref[...], preferred_element_type=jnp.float32)
    cc = jnp.tanh(z + b_ref[0:1, :])
    u = u_ref[...]
    o_ref[...] = u * hx_ref[...] + (1.0 - u) * cc


def kernel(inputs, hx, edge_index, edge_weight, W_fn, b_fn, W_gc, b_gc):
    row = edge_index[0].astype(jnp.int32)
    col = edge_index[1].astype(jnp.int32)
    w = edge_weight.astype(jnp.float32)
    _edge_prep, _spmm = _get_sc_kernels()
    v1, v2 = _edge_prep(row, col, w)
    col2d = col.reshape(E // KC, KC)
    row2d = row.reshape(E // KC, KC)
    v1_2d = v1.reshape(E // KC, KC)
    v2_2d = v2.reshape(E // KC, KC)

    inputs_f = inputs.astype(jnp.float32)
    hx_f = hx.astype(jnp.float32)

    def xlayout(state_bn):
        xi = inputs_f.reshape(B, N, IN_DIM)
        xs = state_bn.reshape(B, N, UNITS)
        x0 = jnp.concatenate([xi, xs], axis=2).transpose(1, 2, 0)
        return x0.reshape(N, NSL, SW).transpose(1, 0, 2)

    def xcat(ys):
        zs = [
            y.reshape(NSL, N, SW // B, B).transpose(3, 1, 0, 2).reshape(B, N, D)
            for y in ys
        ]
        return jnp.stack(zs, axis=2).reshape(B * N, M * D)

    def fold(Wmat):
        Wr = Wmat.astype(jnp.float32).reshape(D, M, -1)
        Wm = jnp.stack(
            [
                Wr[:, 0] - Wr[:, 2],
                Wr[:, 1] - Wr[:, 4],
                2.0 * Wr[:, 2],
                Wr[:, 3],
                2.0 * Wr[:, 4],
            ],
            axis=0,
        )
        return Wm.reshape(M * D, -1)

    Wfn = fold(W_fn)
    Wgc = fold(W_gc)
    bfn = jnp.tile(b_fn.astype(jnp.float32)[None, :], (8, 1))
    bgc = jnp.tile(b_gc.astype(jnp.float32)[None, :], (8, 1))
    hx_bn = hx_f.reshape(B * N, UNITS)

    RB = 1600
    G = (B * N) // RB

    def diffuse(x0):
        y1 = _spmm(x0, col2d, row, v1_2d)
        y2 = _spmm(y1, col2d, row, v1_2d)
        y3 = _spmm(y1, row2d, col, v2_2d)
        y4 = _spmm(y3, row2d, col, v2_2d)
        return xcat([x0, y1, y2, y3, y4])

    xc1 = diffuse(xlayout(hx_f))
    u, rh = pl.pallas_call(
        _mix_sig_body,
        grid=(G,),
        in_specs=[
            pl.BlockSpec((RB, M * D), lambda i: (i, jnp.int32(0))),
            pl.BlockSpec((M * D, 2 * UNITS), lambda i: (jnp.int32(0), jnp.int32(0))),
            pl.BlockSpec((8, 2 * UNITS), lambda i: (jnp.int32(0), jnp.int32(0))),
            pl.BlockSpec((RB, UNITS), lambda i: (i, jnp.int32(0))),
        ],
        out_specs=[pl.BlockSpec((RB, UNITS), lambda i: (i, jnp.int32(0)))] * 2,
        out_shape=[jax.ShapeDtypeStruct((B * N, UNITS), jnp.float32)] * 2,
    )(xc1, Wfn, bfn, hx_bn)

    xc2 = diffuse(xlayout(rh))
    new_state = pl.pallas_call(
        _mix_gate_body,
        grid=(G,),
        in_specs=[
            pl.BlockSpec((RB, M * D), lambda i: (i, jnp.int32(0))),
            pl.BlockSpec((M * D, UNITS), lambda i: (jnp.int32(0), jnp.int32(0))),
            pl.BlockSpec((8, UNITS), lambda i: (jnp.int32(0), jnp.int32(0))),
            pl.BlockSpec((RB, UNITS), lambda i: (i, jnp.int32(0))),
            pl.BlockSpec((RB, UNITS), lambda i: (i, jnp.int32(0))),
        ],
        out_specs=pl.BlockSpec((RB, UNITS), lambda i: (i, jnp.int32(0))),
        out_shape=jax.ShapeDtypeStruct((B * N, UNITS), jnp.float32),
    )(xc2, Wgc, bgc, u, hx_bn)
    return new_state.reshape(B, N * UNITS)


# speedup vs baseline: 2.1270x; 1.1841x over previous
import jax
import jax.numpy as jnp
from jax import lax
from jax.experimental import pallas as pl
from jax.experimental.pallas import tpu as pltpu
from jax.experimental.pallas import tpu_sc as plsc

N = 10000
E = 160000
IN_DIM = 2
UNITS = 64
B = 16
D = IN_DIM + UNITS
M = 5
NSL = 11
SW = (D * B) // NSL
VEC = 16
EPT = E // 16
KC = 200
NCH = EPT // KC
RPT = N // 16
DRC = 125
DEGR = (N + VEC * VEC - 1) // VEC + 1


def _fori(hi, body):
    lax.fori_loop(jnp.int32(0), jnp.int32(hi), lambda i, a: (body(i), a)[1], jnp.int32(0))

def _edge_prep_body(rowh, colh, wh, v1h, v2h, idx_v, w_v, out_v, degr_v, degc_v):
    c = lax.axis_index("c")
    s = lax.axis_index("s")
    wid = s * 2 + c

    def zero_deg(i):
        degr_v[i] = jnp.zeros((VEC,), jnp.float32)
        degc_v[i] = jnp.zeros((VEC,), jnp.float32)

    _fori(640, zero_deg)

    def acc_table(idxh, deg_v):
        def blk(bi):
            pltpu.sync_copy(idxh.at[pl.ds(bi * EPT, EPT)], idx_v)
            pltpu.sync_copy(wh.at[pl.ds(bi * EPT, EPT)], w_v)

            def ed(i):
                o = pl.multiple_of(i * VEC, VEC)
                ids = idx_v[pl.ds(o, VEC)]
                hi = lax.shift_right_logical(ids, jnp.int32(4))
                lo = lax.bitwise_and(ids, jnp.int32(15))
                plsc.addupdate_scatter(deg_v, [hi, lo], w_v[pl.ds(o, VEC)])

            _fori(EPT // VEC, ed)

        _fori(16, blk)

    acc_table(rowh, degr_v)
    acc_table(colh, degc_v)

    def dinv(i):
        dr = degr_v[i]
        degr_v[i] = jnp.where(dr > 0, 1.0 / dr, jnp.zeros((VEC,), jnp.float32))
        dc = degc_v[i]
        degc_v[i] = jnp.where(dc > 0, 1.0 / dc, jnp.zeros((VEC,), jnp.float32))

    _fori(640, dinv)

    def vpass(idxh, deg_v, outh):
        sh = E // 32
        pltpu.sync_copy(idxh.at[pl.ds(wid * sh, sh)], idx_v.at[pl.ds(0, sh)])
        pltpu.sync_copy(wh.at[pl.ds(wid * sh, sh)], w_v.at[pl.ds(0, sh)])

        def ve(i):
            o = pl.multiple_of(i * VEC, VEC)
            ids = idx_v[pl.ds(o, VEC)]
            hi = lax.shift_right_logical(ids, jnp.int32(4))
            lo = lax.bitwise_and(ids, jnp.int32(15))
            dv = plsc.load_gather(deg_v, [hi, lo])
            out_v[pl.ds(o, VEC)] = w_v[pl.ds(o, VEC)] * dv

        _fori((sh + VEC - 1) // VEC, ve)
        pltpu.sync_copy(out_v.at[pl.ds(0, sh)], outh.at[pl.ds(wid * sh, sh)])

    vpass(rowh, degr_v, v1h)
    vpass(colh, degc_v, v2h)


def _build_edge_prep(mesh):
    return pl.kernel(
        _edge_prep_body,
        out_type=(
            jax.ShapeDtypeStruct((E,), jnp.float32),
            jax.ShapeDtypeStruct((E,), jnp.float32),
        ),
        mesh=mesh,
        compiler_params=pltpu.CompilerParams(needs_layout_passes=False, use_tc_tiling_on_sc=False),
        scratch_types=[
            pltpu.VMEM((EPT,), jnp.int32),
            pltpu.VMEM((EPT,), jnp.float32),
            pltpu.VMEM((5008,), jnp.float32),
            pltpu.VMEM((DEGR, VEC), jnp.float32),
            pltpu.VMEM((DEGR, VEC), jnp.float32),
        ],
    )


def _spmm_body(x6, dsth, srch, valsh, y6, cols_v, dst_v, vals_v, rowbuf, big_v, acc, sem):
    c = lax.axis_index("c")
    t = lax.axis_index("s")
    pltpu.sync_copy(srch.at[pl.ds(t * EPT, EPT)], cols_v)
    pltpu.sync_copy(dsth.at[pl.ds(t * NCH, NCH)], dst_v)
    pltpu.sync_copy(valsh.at[pl.ds(t * NCH, NCH)], vals_v)
    for sl in range((NSL + 1) // 2):
        s_idx = c * ((NSL + 1) // 2) + sl
        _spmm_slice(x6, y6, s_idx, t, cols_v, dst_v, vals_v, rowbuf, big_v, acc, sem)


def _spmm_slice(x6, y6, s_idx, t, cols_v, dst_v, vals_v, rowbuf, big_v, acc, sem):
    @pl.when(s_idx < NSL)
    def _():
        def zrow(i):
            for k2 in range(SW // VEC):
                big_v[i, pl.ds(k2 * VEC, VEC)] = jnp.zeros((VEC,), jnp.float32)

        _fori(DRC, zrow)
        for z in range(RPT // DRC):
            pltpu.sync_copy(big_v, acc.at[pl.ds(t * RPT + z * DRC, DRC)])
        plsc.subcore_barrier()

        def chunk(j):
            o = pl.multiple_of(j * KC, KC)
            pltpu.async_copy(
                x6.at[s_idx].at[cols_v.at[pl.ds(o, KC)]], rowbuf, sem
            ).wait()

            def edge(e):
                jv = lax.broadcast(j, (VEC,))
                ev = lax.broadcast(e, (VEC,))
                vs = plsc.load_gather(vals_v, [jv, ev])
                for i2 in range(SW // VEC):
                    rowbuf[e, pl.ds(i2 * VEC, VEC)] = (
                        rowbuf[e, pl.ds(i2 * VEC, VEC)] * vs
                    )

            _fori(KC, edge)
            pltpu.sync_copy(rowbuf, acc.at[dst_v.at[j]], add=True)

        _fori(NCH, chunk)
        plsc.subcore_barrier()
        for z in range(RPT // DRC):
            pltpu.sync_copy(acc.at[pl.ds(t * RPT + z * DRC, DRC)], big_v)
            pltpu.sync_copy(big_v, y6.at[s_idx].at[pl.ds(t * RPT + z * DRC, DRC)])
        plsc.subcore_barrier()


def _build_spmm(mesh):
    return pl.kernel(
        _spmm_body,
        out_type=jax.ShapeDtypeStruct((NSL, N, SW), jnp.float32),
        mesh=mesh,
        compiler_params=pltpu.CompilerParams(needs_layout_passes=False, use_tc_tiling_on_sc=False),
        scratch_types=[
            pltpu.VMEM((EPT,), jnp.int32),
            pltpu.VMEM((NCH, KC), jnp.int32),
            pltpu.VMEM((NCH, KC), jnp.float32),
            pltpu.VMEM((KC, SW), jnp.float32),
            pltpu.VMEM((DRC, SW), jnp.float32),
            pltpu.VMEM_SHARED((N, SW), jnp.float32),
            pltpu.SemaphoreType.DMA,
        ],
    )


_SC_KERNELS = []


def _get_sc_kernels():
    if not _SC_KERNELS:
        mesh = plsc.VectorSubcoreMesh(
            core_axis_name="c", subcore_axis_name="s", num_cores=2, num_subcores=16
        )
        _SC_KERNELS.append((_build_edge_prep(mesh), _build_spmm(mesh)))
    return _SC_KERNELS[0]


def _mix_sig_body(x_ref, w_ref, b_ref, hx_ref, u_ref, rh_ref):
    z = jnp.dot(x_ref[...], w_ref[...], preferred_element_type=jnp.float32)
    v = jax.nn.sigmoid(z + b_ref[0:1, :])
    u_ref[...] = v[:, UNITS:]
    rh_ref[...] = v[:, :UNITS] * hx_ref[...]


def _mix_gate_body(x_ref, w_ref, b_ref, u_ref, hx_ref, o_ref):
    z = jnp.dot(x_ref[...], w_ref[...], preferred_element_type=jnp.float32)
    cc = jnp.tanh(z + b_ref[0:1, :])
    u = u_ref[...]
    o_ref[...] = u * hx_ref[...] + (1.0 - u) * cc


def kernel(inputs, hx, edge_index, edge_weight, W_fn, b_fn, W_gc, b_gc):
    row = edge_index[0].astype(jnp.int32)
    col = edge_index[1].astype(jnp.int32)
    w = edge_weight.astype(jnp.float32)
    _edge_prep, _spmm = _get_sc_kernels()
    v1, v2 = _edge_prep(row, col, w)
    col2d = col.reshape(E // KC, KC)
    row2d = row.reshape(E // KC, KC)
    v1_2d = v1.reshape(E // KC, KC)
    v2_2d = v2.reshape(E // KC, KC)

    inputs_f = inputs.astype(jnp.float32)
    hx_f = hx.astype(jnp.float32)

    def xlayout(state_bn):
        xi = inputs_f.reshape(B, N, IN_DIM)
        xs = state_bn.reshape(B, N, UNITS)
        x0 = jnp.concatenate([xi, xs], axis=2).transpose(1, 2, 0)
        return x0.reshape(N, NSL, SW).transpose(1, 0, 2)

    def xcat(ys):
        zs = [
            y.reshape(NSL, N, SW // B, B).transpose(3, 1, 0, 2).reshape(B, N, D)
            for y in ys
        ]
        return jnp.stack(zs, axis=2).reshape(B * N, M * D)

    def fold(Wmat):
        Wr = Wmat.astype(jnp.float32).reshape(D, M, -1)
        Wm = jnp.stack(
            [
                Wr[:, 0] - Wr[:, 2],
                Wr[:, 1] - Wr[:, 4],
                2.0 * Wr[:, 2],
                Wr[:, 3],
                2.0 * Wr[:, 4],
            ],
            axis=0,
        )
        return Wm.reshape(M * D, -1)

    Wfn = fold(W_fn)
    Wgc = fold(W_gc)
    bfn = jnp.tile(b_fn.astype(jnp.float32)[None, :], (8, 1))
    bgc = jnp.tile(b_gc.astype(jnp.float32)[None, :], (8, 1))
    hx_bn = hx_f.reshape(B * N, UNITS)

    RB = 1600
    G = (B * N) // RB

    def diffuse(x0):
        y1 = _spmm(x0, col2d, row, v1_2d)
        y2 = _spmm(y1, col2d, row, v1_2d)
        y3 = _spmm(y1, row2d, col, v2_2d)
        y4 = _spmm(y3, row2d, col, v2_2d)
        return xcat([x0, y1, y2, y3, y4])

    xc1 = diffuse(xlayout(hx_f))
    u, rh = pl.pallas_call(
        _mix_sig_body,
        grid=(G,),
        in_specs=[
            pl.BlockSpec((RB, M * D), lambda i: (i, jnp.int32(0))),
            pl.BlockSpec((M * D, 2 * UNITS), lambda i: (jnp.int32(0), jnp.int32(0))),
            pl.BlockSpec((8, 2 * UNITS), lambda i: (jnp.int32(0), jnp.int32(0))),
            pl.BlockSpec((RB, UNITS), lambda i: (i, jnp.int32(0))),
        ],
        out_specs=[pl.BlockSpec((RB, UNITS), lambda i: (i, jnp.int32(0)))] * 2,
        out_shape=[jax.ShapeDtypeStruct((B * N, UNITS), jnp.float32)] * 2,
    )(xc1, Wfn, bfn, hx_bn)

    xc2 = diffuse(xlayout(rh))
    new_state = pl.pallas_call(
        _mix_gate_body,
        grid=(G,),
        in_specs=[
            pl.BlockSpec((RB, M * D), lambda i: (i, jnp.int32(0))),
            pl.BlockSpec((M * D, UNITS), lambda i: (jnp.int32(0), jnp.int32(0))),
            pl.BlockSpec((8, UNITS), lambda i: (jnp.int32(0), jnp.int32(0))),
            pl.BlockSpec((RB, UNITS), lambda i: (i, jnp.int32(0))),
            pl.BlockSpec((RB, UNITS), lambda i: (i, jnp.int32(0))),
        ],
        out_specs=pl.BlockSpec((RB, UNITS), lambda i: (i, jnp.int32(0))),
        out_shape=jax.ShapeDtypeStruct((B * N, UNITS), jnp.float32),
    )(xc2, Wgc, bgc, u, hx_bn)
    return new_state.reshape(B, N * UNITS)


# speedup vs baseline: 2.7585x; 1.2969x over previous
import jax
import jax.numpy as jnp
from jax import lax
from jax.experimental import pallas as pl
from jax.experimental.pallas import tpu as pltpu
from jax.experimental.pallas import tpu_sc as plsc

N = 10000
E = 160000
IN_DIM = 2
UNITS = 64
B = 16
D = IN_DIM + UNITS
M = 5
NSL = 11
SW = (D * B) // NSL
VEC = 16
EPT = E // 16
KC = 80
NCH = EPT // KC
RPT = N // 16
DRC = 125
DEGR = (N + VEC * VEC - 1) // VEC + 1


def _fori(hi, body):
    lax.fori_loop(jnp.int32(0), jnp.int32(hi), lambda i, a: (body(i), a)[1], jnp.int32(0))

def _edge_prep_body(rowh, colh, wh, v1h, v2h, idx_v, w_v, out_v, degr_v, degc_v):
    c = lax.axis_index("c")
    s = lax.axis_index("s")
    wid = s * 2 + c

    def zero_deg(i):
        degr_v[i] = jnp.zeros((VEC,), jnp.float32)
        degc_v[i] = jnp.zeros((VEC,), jnp.float32)

    _fori(640, zero_deg)

    def acc_table(idxh, deg_v):
        def blk(bi):
            pltpu.sync_copy(idxh.at[pl.ds(bi * EPT, EPT)], idx_v)
            pltpu.sync_copy(wh.at[pl.ds(bi * EPT, EPT)], w_v)

            def ed(i):
                o = pl.multiple_of(i * VEC, VEC)
                ids = idx_v[pl.ds(o, VEC)]
                hi = lax.shift_right_logical(ids, jnp.int32(4))
                lo = lax.bitwise_and(ids, jnp.int32(15))
                plsc.addupdate_scatter(deg_v, [hi, lo], w_v[pl.ds(o, VEC)])

            _fori(EPT // VEC, ed)

        _fori(16, blk)

    acc_table(rowh, degr_v)
    acc_table(colh, degc_v)

    def dinv(i):
        dr = degr_v[i]
        degr_v[i] = jnp.where(dr > 0, 1.0 / dr, jnp.zeros((VEC,), jnp.float32))
        dc = degc_v[i]
        degc_v[i] = jnp.where(dc > 0, 1.0 / dc, jnp.zeros((VEC,), jnp.float32))

    _fori(640, dinv)

    def vpass(idxh, deg_v, outh):
        sh = E // 32
        pltpu.sync_copy(idxh.at[pl.ds(wid * sh, sh)], idx_v.at[pl.ds(0, sh)])
        pltpu.sync_copy(wh.at[pl.ds(wid * sh, sh)], w_v.at[pl.ds(0, sh)])

        def ve(i):
            o = pl.multiple_of(i * VEC, VEC)
            ids = idx_v[pl.ds(o, VEC)]
            hi = lax.shift_right_logical(ids, jnp.int32(4))
            lo = lax.bitwise_and(ids, jnp.int32(15))
            dv = plsc.load_gather(deg_v, [hi, lo])
            out_v[pl.ds(o, VEC)] = w_v[pl.ds(o, VEC)] * dv

        _fori((sh + VEC - 1) // VEC, ve)
        pltpu.sync_copy(out_v.at[pl.ds(0, sh)], outh.at[pl.ds(wid * sh, sh)])

    vpass(rowh, degr_v, v1h)
    vpass(colh, degc_v, v2h)


def _build_edge_prep(mesh):
    return pl.kernel(
        _edge_prep_body,
        out_type=(
            jax.ShapeDtypeStruct((E,), jnp.float32),
            jax.ShapeDtypeStruct((E,), jnp.float32),
        ),
        mesh=mesh,
        compiler_params=pltpu.CompilerParams(needs_layout_passes=False, use_tc_tiling_on_sc=False),
        scratch_types=[
            pltpu.VMEM((EPT,), jnp.int32),
            pltpu.VMEM((EPT,), jnp.float32),
            pltpu.VMEM((5008,), jnp.float32),
            pltpu.VMEM((DEGR, VEC), jnp.float32),
            pltpu.VMEM((DEGR, VEC), jnp.float32),
        ],
    )


def _spmm_body(
    x6, dsth, srch, valsh, y6, cols_v, dst_v, vals_v, rowbuf0, rowbuf1, big_v, acc, sem0, sem1
):
    c = lax.axis_index("c")
    t = lax.axis_index("s")
    pltpu.sync_copy(srch.at[pl.ds(t * EPT, EPT)], cols_v)
    pltpu.sync_copy(dsth.at[pl.ds(t * NCH, NCH)], dst_v)
    pltpu.sync_copy(valsh.at[pl.ds(t * NCH, NCH)], vals_v)
    for sl in range((NSL + 1) // 2):
        s_idx = c * ((NSL + 1) // 2) + sl
        _spmm_slice(
            x6, y6, s_idx, t, cols_v, dst_v, vals_v, rowbuf0, rowbuf1, big_v, acc, sem0, sem1
        )


def _spmm_slice(
    x6, y6, s_idx, t, cols_v, dst_v, vals_v, rowbuf0, rowbuf1, big_v, acc, sem0, sem1
):
    @pl.when(s_idx < NSL)
    def _():
        def zrow(i):
            for k2 in range(SW // VEC):
                big_v[i, pl.ds(k2 * VEC, VEC)] = jnp.zeros((VEC,), jnp.float32)

        _fori(DRC, zrow)
        for z in range(RPT // DRC):
            pltpu.sync_copy(big_v, acc.at[pl.ds(t * RPT + z * DRC, DRC)])
        plsc.subcore_barrier()

        def gsrc(j):
            o = pl.multiple_of(j * KC, 8)
            return x6.at[s_idx].at[cols_v.at[pl.ds(o, KC)]]

        def scale(buf, j):
            def edge(e):
                jv = lax.broadcast(j, (VEC,))
                ev = lax.broadcast(e, (VEC,))
                vs = plsc.load_gather(vals_v, [jv, ev])
                for i2 in range(SW // VEC):
                    buf[e, pl.ds(i2 * VEC, VEC)] = buf[e, pl.ds(i2 * VEC, VEC)] * vs

            _fori(KC, edge)

        pltpu.async_copy(gsrc(jnp.int32(0)), rowbuf0, sem0)

        def pair(p):
            j0 = p * 2
            j1 = j0 + 1
            pltpu.async_copy(gsrc(j1), rowbuf1, sem1)
            pltpu.make_async_copy(gsrc(j0), rowbuf0, sem0).wait()
            scale(rowbuf0, j0)
            pltpu.sync_copy(rowbuf0, acc.at[dst_v.at[j0]], add=True)
            pltpu.async_copy(gsrc(j0 + 2), rowbuf0, sem0)
            pltpu.make_async_copy(gsrc(j1), rowbuf1, sem1).wait()
            scale(rowbuf1, j1)
            pltpu.sync_copy(rowbuf1, acc.at[dst_v.at[j1]], add=True)

        _fori(NCH // 2, pair)
        jl = jnp.int32(NCH - 1)
        pltpu.make_async_copy(gsrc(jl), rowbuf0, sem0).wait()
        scale(rowbuf0, jl)
        pltpu.sync_copy(rowbuf0, acc.at[dst_v.at[jl]], add=True)
        plsc.subcore_barrier()
        for z in range(RPT // DRC):
            pltpu.sync_copy(acc.at[pl.ds(t * RPT + z * DRC, DRC)], big_v)
            pltpu.sync_copy(big_v, y6.at[s_idx].at[pl.ds(t * RPT + z * DRC, DRC)])
        plsc.subcore_barrier()


def _build_spmm(mesh):
    return pl.kernel(
        _spmm_body,
        out_type=jax.ShapeDtypeStruct((NSL, N, SW), jnp.float32),
        mesh=mesh,
        compiler_params=pltpu.CompilerParams(needs_layout_passes=False, use_tc_tiling_on_sc=False),
        scratch_types=[
            pltpu.VMEM((EPT,), jnp.int32),
            pltpu.VMEM((NCH, KC), jnp.int32),
            pltpu.VMEM((NCH, KC), jnp.float32),
            pltpu.VMEM((KC, SW), jnp.float32),
            pltpu.VMEM((KC, SW), jnp.float32),
            pltpu.VMEM((DRC, SW), jnp.float32),
            pltpu.VMEM_SHARED((N, SW), jnp.float32),
            pltpu.SemaphoreType.DMA,
            pltpu.SemaphoreType.DMA,
        ],
    )


_SC_KERNELS = []


def _get_sc_kernels():
    if not _SC_KERNELS:
        mesh = plsc.VectorSubcoreMesh(
            core_axis_name="c", subcore_axis_name="s", num_cores=2, num_subcores=16
        )
        _SC_KERNELS.append((_build_edge_prep(mesh), _build_spmm(mesh)))
    return _SC_KERNELS[0]


def _mix_sig_body(x_ref, w_ref, b_ref, hx_ref, u_ref, rh_ref):
    z = jnp.dot(x_ref[...], w_ref[...], preferred_element_type=jnp.float32)
    v = jax.nn.sigmoid(z + b_ref[0:1, :])
    u_ref[...] = v[:, UNITS:]
    rh_ref[...] = v[:, :UNITS] * hx_ref[...]


def _mix_gate_body(x_ref, w_ref, b_ref, u_ref, hx_ref, o_ref):
    z = jnp.dot(x_ref[...], w_ref[...], preferred_element_type=jnp.float32)
    cc = jnp.tanh(z + b_ref[0:1, :])
    u = u_ref[...]
    o_ref[...] = u * hx_ref[...] + (1.0 - u) * cc


def kernel(inputs, hx, edge_index, edge_weight, W_fn, b_fn, W_gc, b_gc):
    row = edge_index[0].astype(jnp.int32)
    col = edge_index[1].astype(jnp.int32)
    w = edge_weight.astype(jnp.float32)
    _edge_prep, _spmm = _get_sc_kernels()
    v1, v2 = _edge_prep(row, col, w)
    col2d = col.reshape(E // KC, KC)
    row2d = row.reshape(E // KC, KC)
    v1_2d = v1.reshape(E // KC, KC)
    v2_2d = v2.reshape(E // KC, KC)

    inputs_f = inputs.astype(jnp.float32)
    hx_f = hx.astype(jnp.float32)

    def xlayout(state_bn):
        xi = inputs_f.reshape(B, N, IN_DIM)
        xs = state_bn.reshape(B, N, UNITS)
        x0 = jnp.concatenate([xi, xs], axis=2).transpose(1, 2, 0)
        return x0.reshape(N, NSL, SW).transpose(1, 0, 2)

    def xcat(ys):
        zs = [
            y.reshape(NSL, N, SW // B, B).transpose(3, 1, 0, 2).reshape(B, N, D)
            for y in ys
        ]
        return jnp.stack(zs, axis=2).reshape(B * N, M * D)

    def fold(Wmat):
        Wr = Wmat.astype(jnp.float32).reshape(D, M, -1)
        Wm = jnp.stack(
            [
                Wr[:, 0] - Wr[:, 2],
                Wr[:, 1] - Wr[:, 4],
                2.0 * Wr[:, 2],
                Wr[:, 3],
                2.0 * Wr[:, 4],
            ],
            axis=0,
        )
        return Wm.reshape(M * D, -1)

    Wfn = fold(W_fn)
    Wgc = fold(W_gc)
    bfn = jnp.tile(b_fn.astype(jnp.float32)[None, :], (8, 1))
    bgc = jnp.tile(b_gc.astype(jnp.float32)[None, :], (8, 1))
    hx_bn = hx_f.reshape(B * N, UNITS)

    RB = 1600
    G = (B * N) // RB

    def diffuse(x0):
        y1 = _spmm(x0, col2d, row, v1_2d)
        y2 = _spmm(y1, col2d, row, v1_2d)
        y3 = _spmm(y1, row2d, col, v2_2d)
        y4 = _spmm(y3, row2d, col, v2_2d)
        return xcat([x0, y1, y2, y3, y4])

    xc1 = diffuse(xlayout(hx_f))
    u, rh = pl.pallas_call(
        _mix_sig_body,
        grid=(G,),
        in_specs=[
            pl.BlockSpec((RB, M * D), lambda i: (i, jnp.int32(0))),
            pl.BlockSpec((M * D, 2 * UNITS), lambda i: (jnp.int32(0), jnp.int32(0))),
            pl.BlockSpec((8, 2 * UNITS), lambda i: (jnp.int32(0), jnp.int32(0))),
            pl.BlockSpec((RB, UNITS), lambda i: (i, jnp.int32(0))),
        ],
        out_specs=[pl.BlockSpec((RB, UNITS), lambda i: (i, jnp.int32(0)))] * 2,
        out_shape=[jax.ShapeDtypeStruct((B * N, UNITS), jnp.float32)] * 2,
    )(xc1, Wfn, bfn, hx_bn)

    xc2 = diffuse(xlayout(rh))
    new_state = pl.pallas_call(
        _mix_gate_body,
        grid=(G,),
        in_specs=[
            pl.BlockSpec((RB, M * D), lambda i: (i, jnp.int32(0))),
            pl.BlockSpec((M * D, UNITS), lambda i: (jnp.int32(0), jnp.int32(0))),
            pl.BlockSpec((8, UNITS), lambda i: (jnp.int32(0), jnp.int32(0))),
            pl.BlockSpec((RB, UNITS), lambda i: (i, jnp.int32(0))),
            pl.BlockSpec((RB, UNITS), lambda i: (i, jnp.int32(0))),
        ],
        out_specs=pl.BlockSpec((RB, UNITS), lambda i: (i, jnp.int32(0))),
        out_shape=jax.ShapeDtypeStruct((B * N, UNITS), jnp.float32),
    )(xc2, Wgc, bgc, u, hx_bn)
    return new_state.reshape(B, N * UNITS)
